# baseline (speedup 1.0000x reference)
import functools

import jax
import jax.numpy as jnp
from jax.experimental import pallas as pl
from jax.experimental.pallas import tpu as pltpu

BM = 64


def _router_dispatch_kernel(x_ref, wr_ref, tid_ref, wsl_ref, meta_ref, *, nb):
    T, D = x_ref.shape
    E = wr_ref.shape[0]
    P = 2 * T
    ppad = nb * BM

    tokens = x_ref[...]
    logits = jax.lax.dot_general(
        tokens, wr_ref[...], (((1,), (1,)), ((), ())),
        preferred_element_type=jnp.float32)
    m = jnp.max(logits, axis=-1, keepdims=True)
    ex = jnp.exp(logits - m)
    probs = ex / jnp.sum(ex, axis=-1, keepdims=True)

    eidx = jax.lax.broadcasted_iota(jnp.int32, (T, E), 1)
    i1 = jnp.argmax(probs, axis=-1)[:, None].astype(jnp.int32)
    p1 = jnp.max(probs, axis=-1, keepdims=True)
    probs2 = jnp.where(eidx == i1, -jnp.inf, probs)
    i2 = jnp.argmax(probs2, axis=-1)[:, None].astype(jnp.int32)
    p2 = jnp.max(probs2, axis=-1, keepdims=True)

    e_pairs = jnp.concatenate([i1, i2], axis=0)
    w_pairs = jnp.concatenate([p1, p2], axis=0)
    tl = jax.lax.broadcasted_iota(jnp.int32, (T, 1), 0).astype(jnp.float32)
    t_pairs = jnp.concatenate([tl, tl], axis=0)

    eh = (e_pairs == jax.lax.broadcasted_iota(jnp.int32, (P, E), 1)
          ).astype(jnp.float32)
    counts = jnp.sum(eh, axis=0, keepdims=True)
    ltri = (jax.lax.broadcasted_iota(jnp.int32, (P, P), 1)
            < jax.lax.broadcasted_iota(jnp.int32, (P, P), 0)
            ).astype(jnp.float32)
    ecsum = jax.lax.dot_general(
        ltri, eh, (((1,), (0,)), ((), ())),
        preferred_element_type=jnp.float32)
    rank = jnp.sum(ecsum * eh, axis=1, keepdims=True)

    pc = jnp.ceil(counts / BM) * BM
    etri = (jax.lax.broadcasted_iota(jnp.int32, (E, E), 0)
            < jax.lax.broadcasted_iota(jnp.int32, (E, E), 1)
            ).astype(jnp.float32)
    po = jax.lax.dot_general(
        pc, etri, (((1,), (0,)), ((), ())),
        preferred_element_type=jnp.float32)
    po_p = jax.lax.dot_general(
        eh, po, (((1,), (1,)), ((), ())),
        preferred_element_type=jnp.float32)
    slot = po_p + rank

    S = (slot == jax.lax.broadcasted_iota(jnp.int32, (P, ppad), 1
         ).astype(jnp.float32)).astype(jnp.float32)
    tid_row = jax.lax.dot_general(
        t_pairs, S, (((0,), (0,)), ((), ())),
        preferred_element_type=jnp.float32)
    wsl_row = jax.lax.dot_general(
        w_pairs, S, (((0,), (0,)), ((), ())),
        preferred_element_type=jnp.float32)
    tid_ref[...] = tid_row.astype(jnp.int32)
    wsl_ref[...] = wsl_row

    ends_b = jnp.broadcast_to(po + pc, (nb, E))
    posb = jax.lax.broadcasted_iota(jnp.int32, (nb, E), 0).astype(jnp.float32) * BM
    cnt = jnp.sum((ends_b <= posb).astype(jnp.int32), axis=1,
                  keepdims=True)
    total = jnp.sum(pc)
    e_iota = jax.lax.broadcasted_iota(jnp.int32, (1, E), 1)
    last_e = jnp.max(jnp.where(counts > 0, e_iota, -1))
    posb_col = jax.lax.broadcasted_iota(jnp.int32, (nb, 1), 0).astype(jnp.float32) * BM
    bte = jnp.where(posb_col < total, cnt, last_e)
    nreal = (total / BM).astype(jnp.int32)
    meta_ref[...] = jnp.concatenate(
        [bte, jnp.full((1, 1), nreal, jnp.int32)], axis=0)


def _expert_kernel(meta_ref, x_ref, *rest, nb, nc):
    w1_refs = rest[0:nc]
    w2_refs = rest[nc:2 * nc]
    w3_refs = rest[2 * nc:3 * nc]
    tid_ref, wsl_ref, out_ref = rest[3 * nc:3 * nc + 3]
    b = pl.program_id(0)
    T, D = x_ref.shape

    @pl.when(b == 0)
    def _init():
        out_ref[...] = jnp.zeros_like(out_ref)

    nreal = meta_ref[nb]

    @pl.when(b < nreal)
    def _compute():
        tids = tid_ref[0]
        wsl = wsl_ref[0]
        tiota = jax.lax.broadcasted_iota(jnp.int32, (T, BM), 0)
        gt = (tiota == tids).astype(jnp.float32)
        xb = jax.lax.dot_general(
            gt, x_ref[...], (((0,), (0,)), ((), ())),
            preferred_element_type=jnp.float32
            ).astype(jnp.bfloat16)
        hs = []
        for w1_ref, w2_ref in zip(w1_refs, w2_refs):
            w1 = w1_ref[0, 0].astype(jnp.bfloat16)
            w2 = w2_ref[0, 0].astype(jnp.bfloat16)
            a = jax.lax.dot_general(
                xb, w2, (((1,), (1,)), ((), ())),
                preferred_element_type=jnp.float32)
            g = jax.lax.dot_general(
                xb, w1, (((1,), (1,)), ((), ())),
                preferred_element_type=jnp.float32)
            hs.append(a * jax.nn.sigmoid(a) * g)
        h = jnp.concatenate(hs, axis=1).astype(jnp.bfloat16)
        gw = gt * wsl
        for j, w3_ref in enumerate(w3_refs):
            w3 = w3_ref[0, 0].astype(jnp.bfloat16)
            y = jax.lax.dot_general(
                h, w3, (((1,), (1,)), ((), ())),
                preferred_element_type=jnp.float32)
            dc = w3.shape[0]
            out_ref[:, j * dc:(j + 1) * dc] += jax.lax.dot_general(
                gw, y, (((1,), (0,)), ((), ())),
                preferred_element_type=jnp.float32)


def kernel(x, Wr, W1, W2, W3):
    B, S, D = x.shape
    T = B * S
    E, F, _ = W1.shape
    P = 2 * T
    nb = -(-(P + E * (BM - 1)) // BM)
    ppad = nb * BM
    tokens = x.reshape(T, D)

    tid, wsl, meta = pl.pallas_call(
        functools.partial(_router_dispatch_kernel, nb=nb),
        out_shape=[
            jax.ShapeDtypeStruct((1, ppad), jnp.int32),
            jax.ShapeDtypeStruct((1, ppad), jnp.float32),
            jax.ShapeDtypeStruct((nb + 1, 1), jnp.int32),
        ],
    )(tokens, Wr)

    tid3 = tid.reshape(nb, 1, BM)
    wsl3 = wsl.reshape(nb, 1, BM)
    meta1 = meta.reshape(nb + 1)

    nc = 4
    fc = F // nc
    dc = D // nc
    W1c = W1.reshape(E, nc, fc, D)
    W2c = W2.reshape(E, nc, fc, D)
    W3c = W3.reshape(E, nc, dc, F)

    def wspec(rows, cols, c):
        return pl.BlockSpec((1, 1, rows, cols),
                            lambda b, m, c=c: (m[b], c, 0, 0))

    grid_spec = pltpu.PrefetchScalarGridSpec(
        num_scalar_prefetch=1,
        grid=(nb,),
        in_specs=[
            pl.BlockSpec((T, D), lambda b, m: (0, 0)),
            *[wspec(fc, D, c) for c in range(nc)],
            *[wspec(fc, D, c) for c in range(nc)],
            *[wspec(dc, F, c) for c in range(nc)],
            pl.BlockSpec((1, 1, BM), lambda b, m: (b, 0, 0)),
            pl.BlockSpec((1, 1, BM), lambda b, m: (b, 0, 0)),
        ],
        out_specs=pl.BlockSpec((T, D), lambda b, m: (0, 0)),
    )
    out = pl.pallas_call(
        functools.partial(_expert_kernel, nb=nb, nc=nc),
        grid_spec=grid_spec,
        out_shape=jax.ShapeDtypeStruct((T, D), jnp.float32),
    )(meta1, tokens, *([W1c] * nc), *([W2c] * nc), *([W3c] * nc),
      tid3, wsl3)
    return out.reshape(B, S, D)


# speedup vs baseline: 1.0264x; 1.0264x over previous
import functools

import jax
import jax.numpy as jnp
from jax.experimental import pallas as pl
from jax.experimental.pallas import tpu as pltpu

BM = 128


def _router_dispatch_kernel(x_ref, wr_ref, tid_ref, wsl_ref, meta_ref, *, nb):
    T, D = x_ref.shape
    E = wr_ref.shape[0]
    P = 2 * T
    ppad = nb * BM

    tokens = x_ref[...]
    logits = jax.lax.dot_general(
        tokens, wr_ref[...], (((1,), (1,)), ((), ())),
        preferred_element_type=jnp.float32)
    m = jnp.max(logits, axis=-1, keepdims=True)
    ex = jnp.exp(logits - m)
    probs = ex / jnp.sum(ex, axis=-1, keepdims=True)

    eidx = jax.lax.broadcasted_iota(jnp.int32, (T, E), 1)
    i1 = jnp.argmax(probs, axis=-1)[:, None].astype(jnp.int32)
    p1 = jnp.max(probs, axis=-1, keepdims=True)
    probs2 = jnp.where(eidx == i1, -jnp.inf, probs)
    i2 = jnp.argmax(probs2, axis=-1)[:, None].astype(jnp.int32)
    p2 = jnp.max(probs2, axis=-1, keepdims=True)

    e_pairs = jnp.concatenate([i1, i2], axis=0)
    w_pairs = jnp.concatenate([p1, p2], axis=0)
    tl = jax.lax.broadcasted_iota(jnp.int32, (T, 1), 0).astype(jnp.float32)
    t_pairs = jnp.concatenate([tl, tl], axis=0)

    eh = (e_pairs == jax.lax.broadcasted_iota(jnp.int32, (P, E), 1)
          ).astype(jnp.float32)
    counts = jnp.sum(eh, axis=0, keepdims=True)
    ltri = (jax.lax.broadcasted_iota(jnp.int32, (P, P), 1)
            < jax.lax.broadcasted_iota(jnp.int32, (P, P), 0)
            ).astype(jnp.float32)
    ecsum = jax.lax.dot_general(
        ltri, eh, (((1,), (0,)), ((), ())),
        preferred_element_type=jnp.float32)
    rank = jnp.sum(ecsum * eh, axis=1, keepdims=True)

    pc = jnp.ceil(counts / BM) * BM
    etri = (jax.lax.broadcasted_iota(jnp.int32, (E, E), 0)
            < jax.lax.broadcasted_iota(jnp.int32, (E, E), 1)
            ).astype(jnp.float32)
    po = jax.lax.dot_general(
        pc, etri, (((1,), (0,)), ((), ())),
        preferred_element_type=jnp.float32)
    po_p = jax.lax.dot_general(
        eh, po, (((1,), (1,)), ((), ())),
        preferred_element_type=jnp.float32)
    slot = po_p + rank

    S = (slot == jax.lax.broadcasted_iota(jnp.int32, (P, ppad), 1
         ).astype(jnp.float32)).astype(jnp.float32)
    tid_row = jax.lax.dot_general(
        t_pairs, S, (((0,), (0,)), ((), ())),
        preferred_element_type=jnp.float32)
    wsl_row = jax.lax.dot_general(
        w_pairs, S, (((0,), (0,)), ((), ())),
        preferred_element_type=jnp.float32)
    tid_ref[...] = tid_row.astype(jnp.int32)
    wsl_ref[...] = wsl_row

    ends_b = jnp.broadcast_to(po + pc, (nb, E))
    posb = jax.lax.broadcasted_iota(jnp.int32, (nb, E), 0).astype(jnp.float32) * BM
    cnt = jnp.sum((ends_b <= posb).astype(jnp.int32), axis=1,
                  keepdims=True)
    total = jnp.sum(pc)
    e_iota = jax.lax.broadcasted_iota(jnp.int32, (1, E), 1)
    last_e = jnp.max(jnp.where(counts > 0, e_iota, -1))
    posb_col = jax.lax.broadcasted_iota(jnp.int32, (nb, 1), 0).astype(jnp.float32) * BM
    bte = jnp.where(posb_col < total, cnt, last_e)
    nreal = (total / BM).astype(jnp.int32)
    meta_ref[...] = jnp.concatenate(
        [bte, jnp.full((1, 1), nreal, jnp.int32)], axis=0)


def _expert_kernel(meta_ref, x_ref, *rest, nb, nc):
    w1_refs = rest[0:nc]
    w2_refs = rest[nc:2 * nc]
    w3_refs = rest[2 * nc:3 * nc]
    tid_ref, wsl_ref, out_ref = rest[3 * nc:3 * nc + 3]
    b = pl.program_id(0)
    T, D = x_ref.shape

    @pl.when(b == 0)
    def _init():
        out_ref[...] = jnp.zeros_like(out_ref)

    nreal = meta_ref[nb]

    @pl.when(b < nreal)
    def _compute():
        tids = tid_ref[0]
        wsl = wsl_ref[0]
        tiota = jax.lax.broadcasted_iota(jnp.int32, (T, BM), 0)
        gt = (tiota == tids).astype(jnp.float32)
        xb = jax.lax.dot_general(
            gt, x_ref[...], (((0,), (0,)), ((), ())),
            preferred_element_type=jnp.float32
            ).astype(jnp.bfloat16)
        hs = []
        for w1_ref, w2_ref in zip(w1_refs, w2_refs):
            w1 = w1_ref[0, 0].astype(jnp.bfloat16)
            w2 = w2_ref[0, 0].astype(jnp.bfloat16)
            a = jax.lax.dot_general(
                xb, w2, (((1,), (1,)), ((), ())),
                preferred_element_type=jnp.float32)
            g = jax.lax.dot_general(
                xb, w1, (((1,), (1,)), ((), ())),
                preferred_element_type=jnp.float32)
            hs.append(a * jax.nn.sigmoid(a) * g)
        h = jnp.concatenate(hs, axis=1).astype(jnp.bfloat16)
        gw = gt * wsl
        for j, w3_ref in enumerate(w3_refs):
            w3 = w3_ref[0, 0].astype(jnp.bfloat16)
            y = jax.lax.dot_general(
                h, w3, (((1,), (1,)), ((), ())),
                preferred_element_type=jnp.float32)
            dc = w3.shape[0]
            out_ref[:, j * dc:(j + 1) * dc] += jax.lax.dot_general(
                gw, y, (((1,), (0,)), ((), ())),
                preferred_element_type=jnp.float32)


def kernel(x, Wr, W1, W2, W3):
    B, S, D = x.shape
    T = B * S
    E, F, _ = W1.shape
    P = 2 * T
    nb = -(-(P + E * (BM - 1)) // BM)
    ppad = nb * BM
    tokens = x.reshape(T, D)

    tid, wsl, meta = pl.pallas_call(
        functools.partial(_router_dispatch_kernel, nb=nb),
        out_shape=[
            jax.ShapeDtypeStruct((1, ppad), jnp.int32),
            jax.ShapeDtypeStruct((1, ppad), jnp.float32),
            jax.ShapeDtypeStruct((nb + 1, 1), jnp.int32),
        ],
    )(tokens, Wr)

    tid3 = tid.reshape(nb, 1, BM)
    wsl3 = wsl.reshape(nb, 1, BM)
    meta1 = meta.reshape(nb + 1)

    nc = 4
    fc = F // nc
    dc = D // nc
    W1c = W1.reshape(E, nc, fc, D)
    W2c = W2.reshape(E, nc, fc, D)
    W3c = W3.reshape(E, nc, dc, F)

    def wspec(rows, cols, c):
        return pl.BlockSpec((1, 1, rows, cols),
                            lambda b, m, c=c: (m[b], c, 0, 0))

    grid_spec = pltpu.PrefetchScalarGridSpec(
        num_scalar_prefetch=1,
        grid=(nb,),
        in_specs=[
            pl.BlockSpec((T, D), lambda b, m: (0, 0)),
            *[wspec(fc, D, c) for c in range(nc)],
            *[wspec(fc, D, c) for c in range(nc)],
            *[wspec(dc, F, c) for c in range(nc)],
            pl.BlockSpec((1, 1, BM), lambda b, m: (b, 0, 0)),
            pl.BlockSpec((1, 1, BM), lambda b, m: (b, 0, 0)),
        ],
        out_specs=pl.BlockSpec((T, D), lambda b, m: (0, 0)),
    )
    out = pl.pallas_call(
        functools.partial(_expert_kernel, nb=nb, nc=nc),
        grid_spec=grid_spec,
        out_shape=jax.ShapeDtypeStruct((T, D), jnp.float32),
    )(meta1, tokens, *([W1c] * nc), *([W2c] * nc), *([W3c] * nc),
      tid3, wsl3)
    return out.reshape(B, S, D)
